# initial kernel scaffold (unmeasured)
import numpy as np
import jax
import jax.numpy as jnp
from jax import lax
from jax.experimental import pallas as pl
from jax.experimental.pallas import tpu as pltpu

N_DEV = 32
B_LOC = 2
SQ = 128
D = 512
H_LOC = 4
DH = 64
TOK = B_LOC * SQ
HD_LOC = H_LOC * DH

_inv = 1.0 / (10000.0 ** (np.arange(0, DH, 2) / DH))
_pos = np.arange(SQ)[:, None] * _inv[None, :]
_cos = np.repeat(np.cos(_pos), 2, axis=-1)
_sin = np.repeat(np.sin(_pos), 2, axis=-1)
_COS = np.tile(_cos, (B_LOC, H_LOC)).astype(np.float32)
_SIN = np.tile(_sin, (B_LOC, H_LOC)).astype(np.float32)
_R = np.zeros((DH, DH), dtype=np.float32)
for _k in range(DH // 2):
    _R[2 * _k + 1, 2 * _k] = -1.0
    _R[2 * _k, 2 * _k + 1] = 1.0
_R256 = np.kron(np.eye(H_LOC, dtype=np.float32), _R)


def kernel(x, Wq, Wk, Wv, Wo):
    xs = x.astype(jnp.bfloat16).reshape(TOK, D)
    wq = Wq.astype(jnp.bfloat16)
    wk = Wk.astype(jnp.bfloat16)
    wv = Wv.astype(jnp.bfloat16)
    wo = Wo.astype(jnp.bfloat16)

    def body(x_ref, wq_ref, wk_ref, wv_ref, wo_ref, out_ref,
             xfull, part, comm2, ag_send, ag_recv, rs_send, rs_recv):
        me = lax.axis_index("i")
        right = lax.rem(me + 1, N_DEV)
        left = lax.rem(me + N_DEV - 1, N_DEV)

        barrier_sem = pltpu.get_barrier_semaphore()
        for nbr in (left, right):
            pl.semaphore_signal(
                barrier_sem, inc=1,
                device_id=(nbr,), device_id_type=pl.DeviceIdType.MESH,
            )
        pl.semaphore_wait(barrier_sem, 2)

        xfull[me] = x_ref[:, :]

        for h in range(N_DEV - 1):
            slot = lax.rem(me - h + N_DEV, N_DEV)
            rdma = pltpu.make_async_remote_copy(
                src_ref=xfull.at[slot],
                dst_ref=xfull.at[slot],
                send_sem=ag_send.at[h % 2],
                recv_sem=ag_recv.at[h % 2],
                device_id=(right,),
                device_id_type=pl.DeviceIdType.MESH,
            )
            rdma.start()
            rdma.wait()

        cos_c = jnp.asarray(_COS)
        sin_c = jnp.asarray(_SIN)
        r256 = jnp.asarray(_R256, dtype=jnp.bfloat16)

        def compute_chunk(c, carry):
            xa = xfull[c]
            q = jnp.dot(xa, wq_ref[:, :], preferred_element_type=jnp.float32)
            k = jnp.dot(xa, wk_ref[:, :], preferred_element_type=jnp.float32)
            v = jnp.dot(xa, wv_ref[:, :],
                        preferred_element_type=jnp.float32).astype(jnp.bfloat16)
            qr = jnp.dot(q.astype(jnp.bfloat16), r256,
                         preferred_element_type=jnp.float32)
            kr = jnp.dot(k.astype(jnp.bfloat16), r256,
                         preferred_element_type=jnp.float32)
            qrot = (q * cos_c + qr * sin_c).astype(jnp.bfloat16)
            krot = (k * cos_c + kr * sin_c).astype(jnp.bfloat16)

            rows = []
            for b in range(B_LOC):
                r0 = b * SQ
                ctxs = []
                for hh in range(H_LOC):
                    c0 = hh * DH
                    qh = qrot[r0:r0 + SQ, c0:c0 + DH]
                    kh = krot[r0:r0 + SQ, c0:c0 + DH]
                    vh = v[r0:r0 + SQ, c0:c0 + DH]
                    s = lax.dot_general(
                        qh, kh, (((1,), (1,)), ((), ())),
                        preferred_element_type=jnp.float32) * 0.125
                    m = jnp.max(s, axis=-1, keepdims=True)
                    e = jnp.exp(s - m)
                    w = e / jnp.sum(e, axis=-1, keepdims=True)
                    ctxs.append(jnp.dot(w.astype(jnp.bfloat16), vh,
                                        preferred_element_type=jnp.float32))
                rows.append(jnp.concatenate(ctxs, axis=1))
            ctx = jnp.concatenate(rows, axis=0).astype(jnp.bfloat16)
            part[c] = jnp.dot(ctx, wo_ref[:, :],
                              preferred_element_type=jnp.float32)
            return carry

        lax.fori_loop(0, N_DEV, compute_chunk, 0)

        comm2[0] = part[left]
        for s in range(N_DEV - 1):
            snd = s % 2
            rcv = (s + 1) % 2
            rdma = pltpu.make_async_remote_copy(
                src_ref=comm2.at[snd],
                dst_ref=comm2.at[rcv],
                send_sem=rs_send.at[snd],
                recv_sem=rs_recv.at[rcv],
                device_id=(right,),
                device_id_type=pl.DeviceIdType.MESH,
            )
            rdma.start()
            rdma.wait()
            c_recv = lax.rem(me - s - 2 + 2 * N_DEV, N_DEV)
            if s < N_DEV - 2:
                comm2[rcv] = comm2[rcv] + part[c_recv]
            else:
                out_ref[:, :] = comm2[rcv] + part[me]

    out = pl.pallas_call(
        body,
        out_shape=jax.ShapeDtypeStruct((TOK, D), jnp.float32),
        in_specs=[pl.BlockSpec(memory_space=pltpu.VMEM)] * 5,
        out_specs=pl.BlockSpec(memory_space=pltpu.VMEM),
        scratch_shapes=[
            pltpu.VMEM((N_DEV, TOK, D), jnp.bfloat16),
            pltpu.VMEM((N_DEV, TOK, D), jnp.float32),
            pltpu.VMEM((2, TOK, D), jnp.float32),
            pltpu.SemaphoreType.DMA((2,)),
            pltpu.SemaphoreType.DMA((2,)),
            pltpu.SemaphoreType.DMA((2,)),
            pltpu.SemaphoreType.DMA((2,)),
        ],
        compiler_params=pltpu.CompilerParams(collective_id=0),
    )(xs, wq, wk, wv, wo)
    return out.reshape(B_LOC, SQ, D)


# baseline (device time: 494831 ns/iter reference)
import numpy as np
import jax
import jax.numpy as jnp
from jax import lax
from jax.experimental import pallas as pl
from jax.experimental.pallas import tpu as pltpu

N_DEV = 32
B_LOC = 2
SQ = 128
D = 512
H_LOC = 4
DH = 64
TOK = B_LOC * SQ
HD_LOC = H_LOC * DH

_inv = 1.0 / (10000.0 ** (np.arange(0, DH, 2) / DH))
_pos = np.arange(SQ)[:, None] * _inv[None, :]
_cos = np.repeat(np.cos(_pos), 2, axis=-1)
_sin = np.repeat(np.sin(_pos), 2, axis=-1)
_COS = np.tile(_cos, (B_LOC, H_LOC)).astype(np.float32)
_SIN = np.tile(_sin, (B_LOC, H_LOC)).astype(np.float32)
_R = np.zeros((DH, DH), dtype=np.float32)
for _k in range(DH // 2):
    _R[2 * _k + 1, 2 * _k] = -1.0
    _R[2 * _k, 2 * _k + 1] = 1.0
_R256 = np.kron(np.eye(H_LOC, dtype=np.float32), _R)


def kernel(x, Wq, Wk, Wv, Wo):
    xs = x.astype(jnp.bfloat16).reshape(TOK, D)
    wq = Wq.astype(jnp.bfloat16)
    wk = Wk.astype(jnp.bfloat16)
    wv = Wv.astype(jnp.bfloat16)
    wo = Wo.astype(jnp.bfloat16)

    def body(x_ref, wq_ref, wk_ref, wv_ref, wo_ref, cos_ref, sin_ref, r_ref,
             out_ref, xfull, part, comm2, ag_send, ag_recv, rs_send, rs_recv):
        me = lax.axis_index("i")
        right = lax.rem(me + 1, N_DEV)
        left = lax.rem(me + N_DEV - 1, N_DEV)

        barrier_sem = pltpu.get_barrier_semaphore()
        for nbr in (left, right):
            pl.semaphore_signal(
                barrier_sem, inc=1,
                device_id=(nbr,), device_id_type=pl.DeviceIdType.MESH,
            )
        pl.semaphore_wait(barrier_sem, 2)

        xfull[me] = x_ref[:, :]

        for h in range(N_DEV - 1):
            slot = lax.rem(me - h + N_DEV, N_DEV)
            rdma = pltpu.make_async_remote_copy(
                src_ref=xfull.at[slot],
                dst_ref=xfull.at[slot],
                send_sem=ag_send.at[h % 2],
                recv_sem=ag_recv.at[h % 2],
                device_id=(right,),
                device_id_type=pl.DeviceIdType.MESH,
            )
            rdma.start()
            rdma.wait()

        cos_c = cos_ref[:, :]
        sin_c = sin_ref[:, :]
        r256 = r_ref[:, :]

        def compute_chunk(c, carry):
            xa = xfull[c]
            q = jnp.dot(xa, wq_ref[:, :], preferred_element_type=jnp.float32)
            k = jnp.dot(xa, wk_ref[:, :], preferred_element_type=jnp.float32)
            v = jnp.dot(xa, wv_ref[:, :],
                        preferred_element_type=jnp.float32).astype(jnp.bfloat16)
            qr = jnp.dot(q.astype(jnp.bfloat16), r256,
                         preferred_element_type=jnp.float32)
            kr = jnp.dot(k.astype(jnp.bfloat16), r256,
                         preferred_element_type=jnp.float32)
            qrot = (q * cos_c + qr * sin_c).astype(jnp.bfloat16)
            krot = (k * cos_c + kr * sin_c).astype(jnp.bfloat16)

            rows = []
            for b in range(B_LOC):
                r0 = b * SQ
                ctxs = []
                for hh in range(H_LOC):
                    c0 = hh * DH
                    qh = qrot[r0:r0 + SQ, c0:c0 + DH]
                    kh = krot[r0:r0 + SQ, c0:c0 + DH]
                    vh = v[r0:r0 + SQ, c0:c0 + DH]
                    s = lax.dot_general(
                        qh, kh, (((1,), (1,)), ((), ())),
                        preferred_element_type=jnp.float32) * 0.125
                    m = jnp.max(s, axis=-1, keepdims=True)
                    e = jnp.exp(s - m)
                    w = e / jnp.sum(e, axis=-1, keepdims=True)
                    ctxs.append(jnp.dot(w.astype(jnp.bfloat16), vh,
                                        preferred_element_type=jnp.float32))
                rows.append(jnp.concatenate(ctxs, axis=1))
            ctx = jnp.concatenate(rows, axis=0).astype(jnp.bfloat16)
            part[c] = jnp.dot(ctx, wo_ref[:, :],
                              preferred_element_type=jnp.float32)
            return carry

        lax.fori_loop(0, N_DEV, compute_chunk, 0)

        comm2[0] = part[left]
        for s in range(N_DEV - 1):
            snd = s % 2
            rcv = (s + 1) % 2
            rdma = pltpu.make_async_remote_copy(
                src_ref=comm2.at[snd],
                dst_ref=comm2.at[rcv],
                send_sem=rs_send.at[snd],
                recv_sem=rs_recv.at[rcv],
                device_id=(right,),
                device_id_type=pl.DeviceIdType.MESH,
            )
            rdma.start()
            rdma.wait()
            c_recv = lax.rem(me - s - 2 + 2 * N_DEV, N_DEV)
            if s < N_DEV - 2:
                comm2[rcv] = comm2[rcv] + part[c_recv]
            else:
                out_ref[:, :] = comm2[rcv] + part[me]

    out = pl.pallas_call(
        body,
        out_shape=jax.ShapeDtypeStruct((TOK, D), jnp.float32),
        in_specs=[pl.BlockSpec(memory_space=pltpu.VMEM)] * 8,
        out_specs=pl.BlockSpec(memory_space=pltpu.VMEM),
        scratch_shapes=[
            pltpu.VMEM((N_DEV, TOK, D), jnp.bfloat16),
            pltpu.VMEM((N_DEV, TOK, D), jnp.float32),
            pltpu.VMEM((2, TOK, D), jnp.float32),
            pltpu.SemaphoreType.DMA((2,)),
            pltpu.SemaphoreType.DMA((2,)),
            pltpu.SemaphoreType.DMA((2,)),
            pltpu.SemaphoreType.DMA((2,)),
        ],
        compiler_params=pltpu.CompilerParams(collective_id=0),
    )(xs, wq, wk, wv, wo,
      jnp.asarray(_COS), jnp.asarray(_SIN),
      jnp.asarray(_R256, dtype=jnp.bfloat16))
    return out.reshape(B_LOC, SQ, D)


# device time: 303978 ns/iter; 1.6279x vs baseline; 1.6279x over previous
import numpy as np
import jax
import jax.numpy as jnp
from jax import lax
from jax.experimental import pallas as pl
from jax.experimental.pallas import tpu as pltpu

N_DEV = 32
B_LOC = 2
SQ = 128
D = 512
H_LOC = 4
DH = 64
TOK = B_LOC * SQ
HD_LOC = H_LOC * DH

_inv = 1.0 / (10000.0 ** (np.arange(0, DH, 2) / DH))
_pos = np.arange(SQ)[:, None] * _inv[None, :]
_cos = np.repeat(np.cos(_pos), 2, axis=-1)
_sin = np.repeat(np.sin(_pos), 2, axis=-1)
_COS = np.tile(_cos, (B_LOC, H_LOC)).astype(np.float32)
_SIN = np.tile(_sin, (B_LOC, H_LOC)).astype(np.float32)
_R = np.zeros((DH, DH), dtype=np.float32)
for _k in range(DH // 2):
    _R[2 * _k + 1, 2 * _k] = -1.0
    _R[2 * _k, 2 * _k + 1] = 1.0
_R256 = np.kron(np.eye(H_LOC, dtype=np.float32), _R)


def kernel(x, Wq, Wk, Wv, Wo):
    xs = x.astype(jnp.bfloat16).reshape(TOK, D)
    wq = Wq.astype(jnp.bfloat16)
    wk = Wk.astype(jnp.bfloat16)
    wv = Wv.astype(jnp.bfloat16)
    wo = Wo.astype(jnp.bfloat16)

    def body(x_ref, wq_ref, wk_ref, wv_ref, wo_ref, cos_ref, sin_ref, r_ref,
             out_ref, xfull, part, comm2, ag_send, ag_recv, rs_send, rs_recv):
        me = lax.axis_index("i")
        right = lax.rem(me + 1, N_DEV)
        left = lax.rem(me + N_DEV - 1, N_DEV)

        barrier_sem = pltpu.get_barrier_semaphore()
        for nbr in (left, right):
            pl.semaphore_signal(
                barrier_sem, inc=1,
                device_id=(nbr,), device_id_type=pl.DeviceIdType.MESH,
            )
        pl.semaphore_wait(barrier_sem, 2)

        xfull[me] = x_ref[:, :]

        cos_c = cos_ref[:, :]
        sin_c = sin_ref[:, :]
        r256 = r_ref[:, :]

        def compute_chunk(c, carry):
            xa = xfull[c]
            q = jnp.dot(xa, wq_ref[:, :], preferred_element_type=jnp.float32)
            k = jnp.dot(xa, wk_ref[:, :], preferred_element_type=jnp.float32)
            v = jnp.dot(xa, wv_ref[:, :],
                        preferred_element_type=jnp.float32).astype(jnp.bfloat16)
            qr = jnp.dot(q.astype(jnp.bfloat16), r256,
                         preferred_element_type=jnp.float32)
            kr = jnp.dot(k.astype(jnp.bfloat16), r256,
                         preferred_element_type=jnp.float32)
            qrot = (q * cos_c + qr * sin_c).astype(jnp.bfloat16)
            krot = (k * cos_c + kr * sin_c).astype(jnp.bfloat16)

            rows = []
            for b in range(B_LOC):
                r0 = b * SQ
                ctxs = []
                for hh in range(H_LOC):
                    c0 = hh * DH
                    qh = qrot[r0:r0 + SQ, c0:c0 + DH]
                    kh = krot[r0:r0 + SQ, c0:c0 + DH]
                    vh = v[r0:r0 + SQ, c0:c0 + DH]
                    s = lax.dot_general(
                        qh, kh, (((1,), (1,)), ((), ())),
                        preferred_element_type=jnp.float32) * 0.125
                    m = jnp.max(s, axis=-1, keepdims=True)
                    e = jnp.exp(s - m)
                    w = e / jnp.sum(e, axis=-1, keepdims=True)
                    ctxs.append(jnp.dot(w.astype(jnp.bfloat16), vh,
                                        preferred_element_type=jnp.float32))
                rows.append(jnp.concatenate(ctxs, axis=1))
            ctx = jnp.concatenate(rows, axis=0).astype(jnp.bfloat16)
            part[c] = jnp.dot(ctx, wo_ref[:, :],
                              preferred_element_type=jnp.float32)
            return carry

        def ag_hop(h, carry):
            slot = lax.rem(me - h + 2 * N_DEV, N_DEV)
            rdma = pltpu.make_async_remote_copy(
                src_ref=xfull.at[slot],
                dst_ref=xfull.at[slot],
                send_sem=ag_send.at[lax.rem(h, 2)],
                recv_sem=ag_recv.at[lax.rem(h, 2)],
                device_id=(right,),
                device_id_type=pl.DeviceIdType.MESH,
            )
            rdma.start()
            compute_chunk(slot, 0)
            rdma.wait()
            return carry

        lax.fori_loop(0, N_DEV - 1, ag_hop, 0)
        compute_chunk(lax.rem(me + 1, N_DEV), 0)

        comm2[0] = part[left].astype(jnp.bfloat16)
        for s in range(N_DEV - 1):
            snd = s % 2
            rcv = (s + 1) % 2
            rdma = pltpu.make_async_remote_copy(
                src_ref=comm2.at[snd],
                dst_ref=comm2.at[rcv],
                send_sem=rs_send.at[snd],
                recv_sem=rs_recv.at[rcv],
                device_id=(right,),
                device_id_type=pl.DeviceIdType.MESH,
            )
            rdma.start()
            rdma.wait()
            c_recv = lax.rem(me - s - 2 + 2 * N_DEV, N_DEV)
            if s < N_DEV - 2:
                comm2[rcv] = (comm2[rcv] + part[c_recv]).astype(jnp.bfloat16)
            else:
                out_ref[:, :] = comm2[rcv] + part[me]

    out = pl.pallas_call(
        body,
        out_shape=jax.ShapeDtypeStruct((TOK, D), jnp.float32),
        in_specs=[pl.BlockSpec(memory_space=pltpu.VMEM)] * 8,
        out_specs=pl.BlockSpec(memory_space=pltpu.VMEM),
        scratch_shapes=[
            pltpu.VMEM((N_DEV, TOK, D), jnp.bfloat16),
            pltpu.VMEM((N_DEV, TOK, D), jnp.float32),
            pltpu.VMEM((2, TOK, D), jnp.bfloat16),
            pltpu.SemaphoreType.DMA((2,)),
            pltpu.SemaphoreType.DMA((2,)),
            pltpu.SemaphoreType.DMA((2,)),
            pltpu.SemaphoreType.DMA((2,)),
        ],
        compiler_params=pltpu.CompilerParams(collective_id=0),
    )(xs, wq, wk, wv, wo,
      jnp.asarray(_COS), jnp.asarray(_SIN),
      jnp.asarray(_R256, dtype=jnp.bfloat16))
    return out.reshape(B_LOC, SQ, D)


# device time: 238536 ns/iter; 2.0744x vs baseline; 1.2743x over previous
import numpy as np
import jax
import jax.numpy as jnp
from jax import lax
from jax.experimental import pallas as pl
from jax.experimental.pallas import tpu as pltpu

N_DEV = 32
HALF = N_DEV // 2
B_LOC = 2
SQ = 128
D = 512
H_LOC = 4
DH = 64
TOK = B_LOC * SQ
HD_LOC = H_LOC * DH

_inv = 1.0 / (10000.0 ** (np.arange(0, DH, 2) / DH))
_pos = np.arange(SQ)[:, None] * _inv[None, :]
_cos = np.repeat(np.cos(_pos), 2, axis=-1)
_sin = np.repeat(np.sin(_pos), 2, axis=-1)
_COS = np.tile(_cos, (B_LOC, H_LOC)).astype(np.float32)
_SIN = np.tile(_sin, (B_LOC, H_LOC)).astype(np.float32)
_R = np.zeros((DH, DH), dtype=np.float32)
for _k in range(DH // 2):
    _R[2 * _k + 1, 2 * _k] = -1.0
    _R[2 * _k, 2 * _k + 1] = 1.0
_R256 = np.kron(np.eye(H_LOC, dtype=np.float32), _R)


def kernel(x, Wq, Wk, Wv, Wo):
    xs = x.astype(jnp.bfloat16).reshape(TOK, D)
    wq = Wq.astype(jnp.bfloat16)
    wk = Wk.astype(jnp.bfloat16)
    wv = Wv.astype(jnp.bfloat16)
    wo = Wo.astype(jnp.bfloat16)

    def body(x_ref, wq_ref, wk_ref, wv_ref, wo_ref, cos_ref, sin_ref, r_ref,
             out_ref, xfull, part, comm_r, comm_l,
             ag_send_r, ag_recv_r, ag_send_l, ag_recv_l,
             rs_send_r, rs_recv_r, rs_send_l, rs_recv_l):
        me = lax.axis_index("i")
        right = lax.rem(me + 1, N_DEV)
        left = lax.rem(me + N_DEV - 1, N_DEV)

        def mod(v):
            return lax.rem(v + 2 * N_DEV, N_DEV)

        barrier_sem = pltpu.get_barrier_semaphore()
        for nbr in (left, right):
            pl.semaphore_signal(
                barrier_sem, inc=1,
                device_id=(nbr,), device_id_type=pl.DeviceIdType.MESH,
            )
        pl.semaphore_wait(barrier_sem, 2)

        xfull[me] = x_ref[:, :]

        cos_c = cos_ref[:, :]
        sin_c = sin_ref[:, :]
        r256 = r_ref[:, :]

        def compute_chunk(c):
            xa = xfull[c]
            q = jnp.dot(xa, wq_ref[:, :], preferred_element_type=jnp.float32)
            k = jnp.dot(xa, wk_ref[:, :], preferred_element_type=jnp.float32)
            v = jnp.dot(xa, wv_ref[:, :],
                        preferred_element_type=jnp.float32).astype(jnp.bfloat16)
            qr = jnp.dot(q.astype(jnp.bfloat16), r256,
                         preferred_element_type=jnp.float32)
            kr = jnp.dot(k.astype(jnp.bfloat16), r256,
                         preferred_element_type=jnp.float32)
            qrot = (q * cos_c + qr * sin_c).astype(jnp.bfloat16)
            krot = (k * cos_c + kr * sin_c).astype(jnp.bfloat16)

            rows = []
            for b in range(B_LOC):
                r0 = b * SQ
                ctxs = []
                for hh in range(H_LOC):
                    c0 = hh * DH
                    qh = qrot[r0:r0 + SQ, c0:c0 + DH]
                    kh = krot[r0:r0 + SQ, c0:c0 + DH]
                    vh = v[r0:r0 + SQ, c0:c0 + DH]
                    s = lax.dot_general(
                        qh, kh, (((1,), (1,)), ((), ())),
                        preferred_element_type=jnp.float32) * 0.125
                    m = jnp.max(s, axis=-1, keepdims=True)
                    e = jnp.exp(s - m)
                    w = e / jnp.sum(e, axis=-1, keepdims=True)
                    ctxs.append(jnp.dot(w.astype(jnp.bfloat16), vh,
                                        preferred_element_type=jnp.float32))
                rows.append(jnp.concatenate(ctxs, axis=1))
            ctx = jnp.concatenate(rows, axis=0).astype(jnp.bfloat16)
            part[c] = jnp.dot(ctx, wo_ref[:, :],
                              preferred_element_type=jnp.float32)

        def ag_rdma_right(h):
            slot = mod(me - h)
            return pltpu.make_async_remote_copy(
                src_ref=xfull.at[slot], dst_ref=xfull.at[slot],
                send_sem=ag_send_r.at[lax.rem(h, 2)],
                recv_sem=ag_recv_r.at[lax.rem(h, 2)],
                device_id=(right,), device_id_type=pl.DeviceIdType.MESH,
            )

        def ag_rdma_left(h):
            slot = mod(me + h)
            return pltpu.make_async_remote_copy(
                src_ref=xfull.at[slot], dst_ref=xfull.at[slot],
                send_sem=ag_send_l.at[lax.rem(h, 2)],
                recv_sem=ag_recv_l.at[lax.rem(h, 2)],
                device_id=(left,), device_id_type=pl.DeviceIdType.MESH,
            )

        r0 = ag_rdma_right(0)
        l0 = ag_rdma_left(0)
        r0.start()
        l0.start()
        compute_chunk(me)
        r0.wait()
        l0.wait()

        def ag_hop(h, carry):
            rr = ag_rdma_right(h)
            ll = ag_rdma_left(h)
            rr.start()
            ll.start()
            compute_chunk(mod(me - h))
            compute_chunk(mod(me + h))
            rr.wait()
            ll.wait()
            return carry

        lax.fori_loop(1, HALF - 1, ag_hop, 0)

        r15 = ag_rdma_right(HALF - 1)
        r15.start()
        compute_chunk(mod(me - (HALF - 1)))
        compute_chunk(mod(me + (HALF - 1)))
        r15.wait()
        compute_chunk(mod(me - HALF))

        comm_r[0] = part[mod(me + HALF)].astype(jnp.bfloat16)
        comm_l[0] = part[mod(me - (HALF - 1))].astype(jnp.bfloat16)

        for s in range(HALF):
            snd = s % 2
            rcv = (s + 1) % 2
            rr = pltpu.make_async_remote_copy(
                src_ref=comm_r.at[snd], dst_ref=comm_r.at[rcv],
                send_sem=rs_send_r.at[snd], recv_sem=rs_recv_r.at[rcv],
                device_id=(right,), device_id_type=pl.DeviceIdType.MESH,
            )
            rr.start()
            if s < HALF - 1:
                ll = pltpu.make_async_remote_copy(
                    src_ref=comm_l.at[snd], dst_ref=comm_l.at[rcv],
                    send_sem=rs_send_l.at[snd], recv_sem=rs_recv_l.at[rcv],
                    device_id=(left,), device_id_type=pl.DeviceIdType.MESH,
                )
                ll.start()
                ll.wait()
                if s < HALF - 2:
                    comm_l[rcv] = (comm_l[rcv]
                                   + part[mod(me - 14 + s)]).astype(jnp.bfloat16)
            rr.wait()
            if s < HALF - 1:
                comm_r[rcv] = (comm_r[rcv]
                               + part[mod(me + 15 - s)]).astype(jnp.bfloat16)

        out_ref[:, :] = comm_r[0] + comm_l[1] + part[me]

    out = pl.pallas_call(
        body,
        out_shape=jax.ShapeDtypeStruct((TOK, D), jnp.float32),
        in_specs=[pl.BlockSpec(memory_space=pltpu.VMEM)] * 8,
        out_specs=pl.BlockSpec(memory_space=pltpu.VMEM),
        scratch_shapes=[
            pltpu.VMEM((N_DEV, TOK, D), jnp.bfloat16),
            pltpu.VMEM((N_DEV, TOK, D), jnp.float32),
            pltpu.VMEM((2, TOK, D), jnp.bfloat16),
            pltpu.VMEM((2, TOK, D), jnp.bfloat16),
            pltpu.SemaphoreType.DMA((2,)),
            pltpu.SemaphoreType.DMA((2,)),
            pltpu.SemaphoreType.DMA((2,)),
            pltpu.SemaphoreType.DMA((2,)),
            pltpu.SemaphoreType.DMA((2,)),
            pltpu.SemaphoreType.DMA((2,)),
            pltpu.SemaphoreType.DMA((2,)),
            pltpu.SemaphoreType.DMA((2,)),
        ],
        compiler_params=pltpu.CompilerParams(collective_id=0),
    )(xs, wq, wk, wv, wo,
      jnp.asarray(_COS), jnp.asarray(_SIN),
      jnp.asarray(_R256, dtype=jnp.bfloat16))
    return out.reshape(B_LOC, SQ, D)


# device time: 234891 ns/iter; 2.1066x vs baseline; 1.0155x over previous
import numpy as np
import jax
import jax.numpy as jnp
from jax import lax
from jax.experimental import pallas as pl
from jax.experimental.pallas import tpu as pltpu

N_DEV = 32
HALF = N_DEV // 2
B_LOC = 2
SQ = 128
D = 512
H_LOC = 4
DH = 64
TOK = B_LOC * SQ
PAIR = 2 * TOK

_inv = 1.0 / (10000.0 ** (np.arange(0, DH, 2) / DH))
_pos = np.arange(SQ)[:, None] * _inv[None, :]
_cos = np.repeat(np.cos(_pos), 2, axis=-1)
_sin = np.repeat(np.sin(_pos), 2, axis=-1)
_COS = np.tile(_cos, (2 * B_LOC, H_LOC)).astype(np.float32)
_SIN = np.tile(_sin, (2 * B_LOC, H_LOC)).astype(np.float32)
_R = np.zeros((DH, DH), dtype=np.float32)
for _k in range(DH // 2):
    _R[2 * _k + 1, 2 * _k] = -1.0
    _R[2 * _k, 2 * _k + 1] = 1.0
_R256 = np.kron(np.eye(H_LOC, dtype=np.float32), _R)
_MASK = np.full((PAIR, PAIR), -1e9, dtype=np.float32)
for _b in range(2 * B_LOC):
    _MASK[_b * SQ:(_b + 1) * SQ, _b * SQ:(_b + 1) * SQ] = 0.0


def kernel(x, Wq, Wk, Wv, Wo):
    xs = x.astype(jnp.bfloat16).reshape(TOK, D)
    wq = Wq.astype(jnp.bfloat16)
    wk = Wk.astype(jnp.bfloat16)
    wv = Wv.astype(jnp.bfloat16)
    wo = Wo.astype(jnp.bfloat16)

    def body(x_ref, wq_ref, wk_ref, wv_ref, wo_ref, cos_ref, sin_ref, r_ref,
             mask_ref, out_ref, xfull, part, comm_r, comm_l,
             ag_send_r, ag_recv_r, ag_send_l, ag_recv_l,
             rs_send_r, rs_recv_r, rs_send_l, rs_recv_l):
        me = lax.axis_index("i")
        right = lax.rem(me + 1, N_DEV)
        left = lax.rem(me + N_DEV - 1, N_DEV)

        def mod(v):
            return lax.rem(v + 2 * N_DEV, N_DEV)

        barrier_sem = pltpu.get_barrier_semaphore()
        for nbr in (left, right):
            pl.semaphore_signal(
                barrier_sem, inc=1,
                device_id=(nbr,), device_id_type=pl.DeviceIdType.MESH,
            )
        pl.semaphore_wait(barrier_sem, 2)

        xfull[me] = x_ref[:, :]

        cos_c = cos_ref[:, :]
        sin_c = sin_ref[:, :]
        r256 = r_ref[:, :]
        mask_c = mask_ref[:, :]

        def attention(xa, n_rows, cosv, sinv, maskv):
            q = jnp.dot(xa, wq_ref[:, :], preferred_element_type=jnp.float32)
            k = jnp.dot(xa, wk_ref[:, :], preferred_element_type=jnp.float32)
            v = jnp.dot(xa, wv_ref[:, :],
                        preferred_element_type=jnp.float32).astype(jnp.bfloat16)
            qr = jnp.dot(q.astype(jnp.bfloat16), r256,
                         preferred_element_type=jnp.float32)
            kr = jnp.dot(k.astype(jnp.bfloat16), r256,
                         preferred_element_type=jnp.float32)
            qrot = (q * cosv + qr * sinv).astype(jnp.bfloat16)
            krot = (k * cosv + kr * sinv).astype(jnp.bfloat16)
            ctxs = []
            for hh in range(H_LOC):
                c0 = hh * DH
                qh = qrot[:, c0:c0 + DH]
                kh = krot[:, c0:c0 + DH]
                vh = v[:, c0:c0 + DH]
                s = lax.dot_general(
                    qh, kh, (((1,), (1,)), ((), ())),
                    preferred_element_type=jnp.float32) * 0.125 + maskv
                e = jnp.exp(s)
                w = e / jnp.sum(e, axis=-1, keepdims=True)
                ctxs.append(jnp.dot(w.astype(jnp.bfloat16), vh,
                                    preferred_element_type=jnp.float32))
            ctx = jnp.concatenate(ctxs, axis=1).astype(jnp.bfloat16)
            return jnp.dot(ctx, wo_ref[:, :], preferred_element_type=jnp.float32)

        def compute_one(c):
            part[c] = attention(xfull[c], TOK, cos_c[:TOK], sin_c[:TOK],
                                mask_c[:TOK, :TOK])

        def compute_pair(c1, c2):
            xa = jnp.concatenate([xfull[c1], xfull[c2]], axis=0)
            res = attention(xa, PAIR, cos_c, sin_c, mask_c)
            part[c1] = res[:TOK]
            part[c2] = res[TOK:]

        def ag_rdma_right(h):
            slot = mod(me - h)
            return pltpu.make_async_remote_copy(
                src_ref=xfull.at[slot], dst_ref=xfull.at[slot],
                send_sem=ag_send_r.at[lax.rem(h, 2)],
                recv_sem=ag_recv_r.at[lax.rem(h, 2)],
                device_id=(right,), device_id_type=pl.DeviceIdType.MESH,
            )

        def ag_rdma_left(h):
            slot = mod(me + h)
            return pltpu.make_async_remote_copy(
                src_ref=xfull.at[slot], dst_ref=xfull.at[slot],
                send_sem=ag_send_l.at[lax.rem(h, 2)],
                recv_sem=ag_recv_l.at[lax.rem(h, 2)],
                device_id=(left,), device_id_type=pl.DeviceIdType.MESH,
            )

        r0 = ag_rdma_right(0)
        l0 = ag_rdma_left(0)
        r0.start()
        l0.start()
        compute_one(me)
        r0.wait()
        l0.wait()

        def ag_hop(h, carry):
            rr = ag_rdma_right(h)
            ll = ag_rdma_left(h)
            rr.start()
            ll.start()
            compute_pair(mod(me - h), mod(me + h))
            rr.wait()
            ll.wait()
            return carry

        lax.fori_loop(1, HALF - 1, ag_hop, 0)

        r15 = ag_rdma_right(HALF - 1)
        r15.start()
        compute_pair(mod(me - (HALF - 1)), mod(me + HALF - 1))
        r15.wait()
        compute_one(mod(me - HALF))

        comm_r[0] = part[mod(me + HALF)].astype(jnp.bfloat16)
        comm_l[0] = part[mod(me - (HALF - 1))].astype(jnp.bfloat16)

        for s in range(HALF):
            snd = s % 2
            rcv = (s + 1) % 2
            rr = pltpu.make_async_remote_copy(
                src_ref=comm_r.at[snd], dst_ref=comm_r.at[rcv],
                send_sem=rs_send_r.at[snd], recv_sem=rs_recv_r.at[rcv],
                device_id=(right,), device_id_type=pl.DeviceIdType.MESH,
            )
            rr.start()
            if s < HALF - 1:
                ll = pltpu.make_async_remote_copy(
                    src_ref=comm_l.at[snd], dst_ref=comm_l.at[rcv],
                    send_sem=rs_send_l.at[snd], recv_sem=rs_recv_l.at[rcv],
                    device_id=(left,), device_id_type=pl.DeviceIdType.MESH,
                )
                ll.start()
                ll.wait()
                if s < HALF - 2:
                    comm_l[rcv] = (comm_l[rcv]
                                   + part[mod(me - 14 + s)]).astype(jnp.bfloat16)
            rr.wait()
            if s < HALF - 1:
                comm_r[rcv] = (comm_r[rcv]
                               + part[mod(me + 15 - s)]).astype(jnp.bfloat16)

        out_ref[:, :] = comm_r[0] + comm_l[1] + part[me]

    out = pl.pallas_call(
        body,
        out_shape=jax.ShapeDtypeStruct((TOK, D), jnp.float32),
        in_specs=[pl.BlockSpec(memory_space=pltpu.VMEM)] * 9,
        out_specs=pl.BlockSpec(memory_space=pltpu.VMEM),
        scratch_shapes=[
            pltpu.VMEM((N_DEV, TOK, D), jnp.bfloat16),
            pltpu.VMEM((N_DEV, TOK, D), jnp.float32),
            pltpu.VMEM((2, TOK, D), jnp.bfloat16),
            pltpu.VMEM((2, TOK, D), jnp.bfloat16),
            pltpu.SemaphoreType.DMA((2,)),
            pltpu.SemaphoreType.DMA((2,)),
            pltpu.SemaphoreType.DMA((2,)),
            pltpu.SemaphoreType.DMA((2,)),
            pltpu.SemaphoreType.DMA((2,)),
            pltpu.SemaphoreType.DMA((2,)),
            pltpu.SemaphoreType.DMA((2,)),
            pltpu.SemaphoreType.DMA((2,)),
        ],
        compiler_params=pltpu.CompilerParams(collective_id=0),
    )(xs, wq, wk, wv, wo,
      jnp.asarray(_COS), jnp.asarray(_SIN),
      jnp.asarray(_R256, dtype=jnp.bfloat16), jnp.asarray(_MASK))
    return out.reshape(B_LOC, SQ, D)


# device time: 157781 ns/iter; 3.1362x vs baseline; 1.4887x over previous
import numpy as np
import jax
import jax.numpy as jnp
from jax import lax
from jax.experimental import pallas as pl
from jax.experimental.pallas import tpu as pltpu

N_DEV = 32
HALF = N_DEV // 2
B_LOC = 2
SQ = 128
D = 512
H_LOC = 4
DH = 64
TOK = B_LOC * SQ
PAIR = 2 * TOK

_inv = 1.0 / (10000.0 ** (np.arange(0, DH, 2) / DH))
_pos = np.arange(SQ)[:, None] * _inv[None, :]
_cos = np.repeat(np.cos(_pos), 2, axis=-1)
_sin = np.repeat(np.sin(_pos), 2, axis=-1)
_COS = np.tile(_cos, (2 * B_LOC, H_LOC)).astype(np.float32)
_SIN = np.tile(_sin, (2 * B_LOC, H_LOC)).astype(np.float32)
_R = np.zeros((DH, DH), dtype=np.float32)
for _k in range(DH // 2):
    _R[2 * _k + 1, 2 * _k] = -1.0
    _R[2 * _k, 2 * _k + 1] = 1.0
_R256 = np.kron(np.eye(H_LOC, dtype=np.float32), _R)
_MASK = np.full((PAIR, PAIR), -1e9, dtype=np.float32)
for _b in range(2 * B_LOC):
    _MASK[_b * SQ:(_b + 1) * SQ, _b * SQ:(_b + 1) * SQ] = 0.0

def _logical(x, y, z):
    return z * 8 + 2 * y + (x if y % 2 == 0 else 1 - x)

_w = []
for _y in range(4):
    for _z in (range(4) if _y % 2 == 0 else range(3, -1, -1)):
        _w.append((_y, _z))
_cycle = [(0, y, z) for (y, z) in _w] + [(1, y, z) for (y, z) in reversed(_w)]
_PERM = np.array([_logical(x, y, z) for (x, y, z) in _cycle], dtype=np.int32)
_RANK = np.zeros(N_DEV, dtype=np.int32)
for _r, _p in enumerate(_PERM):
    _RANK[_p] = _r


def kernel(x, Wq, Wk, Wv, Wo):
    xs = x.astype(jnp.bfloat16).reshape(TOK, D)
    wq = Wq.astype(jnp.bfloat16)
    wk = Wk.astype(jnp.bfloat16)
    wv = Wv.astype(jnp.bfloat16)
    wo = Wo.astype(jnp.bfloat16)

    def body(x_ref, wq_ref, wk_ref, wv_ref, wo_ref, cos_ref, sin_ref, r_ref,
             mask_ref, perm_ref, rank_ref, out_ref, xfull, part, comm_r, comm_l,
             ag_send_r, ag_recv_r, ag_send_l, ag_recv_l,
             rs_send_r, rs_recv_r, rs_send_l, rs_recv_l):
        def mod(v):
            return lax.rem(v + 2 * N_DEV, N_DEV)

        me = rank_ref[lax.axis_index("i")]
        right = perm_ref[mod(me + 1)]
        left = perm_ref[mod(me - 1)]

        barrier_sem = pltpu.get_barrier_semaphore()
        for nbr in (left, right):
            pl.semaphore_signal(
                barrier_sem, inc=1,
                device_id=(nbr,), device_id_type=pl.DeviceIdType.MESH,
            )
        pl.semaphore_wait(barrier_sem, 2)

        xfull[me] = x_ref[:, :]

        cos_c = cos_ref[:, :]
        sin_c = sin_ref[:, :]
        r256 = r_ref[:, :]
        mask_c = mask_ref[:, :]

        def attention(xa, n_rows, cosv, sinv, maskv):
            q = jnp.dot(xa, wq_ref[:, :], preferred_element_type=jnp.float32)
            k = jnp.dot(xa, wk_ref[:, :], preferred_element_type=jnp.float32)
            v = jnp.dot(xa, wv_ref[:, :],
                        preferred_element_type=jnp.float32).astype(jnp.bfloat16)
            qr = jnp.dot(q.astype(jnp.bfloat16), r256,
                         preferred_element_type=jnp.float32)
            kr = jnp.dot(k.astype(jnp.bfloat16), r256,
                         preferred_element_type=jnp.float32)
            qrot = (q * cosv + qr * sinv).astype(jnp.bfloat16)
            krot = (k * cosv + kr * sinv).astype(jnp.bfloat16)
            ctxs = []
            for hh in range(H_LOC):
                c0 = hh * DH
                qh = qrot[:, c0:c0 + DH]
                kh = krot[:, c0:c0 + DH]
                vh = v[:, c0:c0 + DH]
                s = lax.dot_general(
                    qh, kh, (((1,), (1,)), ((), ())),
                    preferred_element_type=jnp.float32) * 0.125 + maskv
                e = jnp.exp(s)
                w = e / jnp.sum(e, axis=-1, keepdims=True)
                ctxs.append(jnp.dot(w.astype(jnp.bfloat16), vh,
                                    preferred_element_type=jnp.float32))
            ctx = jnp.concatenate(ctxs, axis=1).astype(jnp.bfloat16)
            return jnp.dot(ctx, wo_ref[:, :], preferred_element_type=jnp.float32)

        def compute_one(c):
            part[c] = attention(xfull[c], TOK, cos_c[:TOK], sin_c[:TOK],
                                mask_c[:TOK, :TOK])

        def compute_pair(c1, c2):
            xa = jnp.concatenate([xfull[c1], xfull[c2]], axis=0)
            res = attention(xa, PAIR, cos_c, sin_c, mask_c)
            part[c1] = res[:TOK]
            part[c2] = res[TOK:]

        def ag_rdma_right(h):
            slot = mod(me - h)
            return pltpu.make_async_remote_copy(
                src_ref=xfull.at[slot], dst_ref=xfull.at[slot],
                send_sem=ag_send_r.at[lax.rem(h, 2)],
                recv_sem=ag_recv_r.at[lax.rem(h, 2)],
                device_id=(right,), device_id_type=pl.DeviceIdType.MESH,
            )

        def ag_rdma_left(h):
            slot = mod(me + h)
            return pltpu.make_async_remote_copy(
                src_ref=xfull.at[slot], dst_ref=xfull.at[slot],
                send_sem=ag_send_l.at[lax.rem(h, 2)],
                recv_sem=ag_recv_l.at[lax.rem(h, 2)],
                device_id=(left,), device_id_type=pl.DeviceIdType.MESH,
            )

        r0 = ag_rdma_right(0)
        l0 = ag_rdma_left(0)
        r0.start()
        l0.start()
        compute_one(me)
        r0.wait()
        l0.wait()

        def ag_hop(h, carry):
            rr = ag_rdma_right(h)
            ll = ag_rdma_left(h)
            rr.start()
            ll.start()
            compute_pair(mod(me - h), mod(me + h))
            rr.wait()
            ll.wait()
            return carry

        lax.fori_loop(1, HALF - 1, ag_hop, 0)

        r15 = ag_rdma_right(HALF - 1)
        r15.start()
        compute_pair(mod(me - (HALF - 1)), mod(me + HALF - 1))
        r15.wait()
        compute_one(mod(me - HALF))

        comm_r[0] = part[mod(me + HALF)].astype(jnp.bfloat16)
        comm_l[0] = part[mod(me - (HALF - 1))].astype(jnp.bfloat16)

        for s in range(HALF):
            snd = s % 2
            rcv = (s + 1) % 2
            rr = pltpu.make_async_remote_copy(
                src_ref=comm_r.at[snd], dst_ref=comm_r.at[rcv],
                send_sem=rs_send_r.at[snd], recv_sem=rs_recv_r.at[rcv],
                device_id=(right,), device_id_type=pl.DeviceIdType.MESH,
            )
            rr.start()
            if s < HALF - 1:
                ll = pltpu.make_async_remote_copy(
                    src_ref=comm_l.at[snd], dst_ref=comm_l.at[rcv],
                    send_sem=rs_send_l.at[snd], recv_sem=rs_recv_l.at[rcv],
                    device_id=(left,), device_id_type=pl.DeviceIdType.MESH,
                )
                ll.start()
                ll.wait()
                if s < HALF - 2:
                    comm_l[rcv] = (comm_l[rcv]
                                   + part[mod(me - 14 + s)]).astype(jnp.bfloat16)
            rr.wait()
            if s < HALF - 1:
                comm_r[rcv] = (comm_r[rcv]
                               + part[mod(me + 15 - s)]).astype(jnp.bfloat16)

        out_ref[:, :] = comm_r[0] + comm_l[1] + part[me]

    out = pl.pallas_call(
        body,
        out_shape=jax.ShapeDtypeStruct((TOK, D), jnp.float32),
        in_specs=([pl.BlockSpec(memory_space=pltpu.VMEM)] * 9
                  + [pl.BlockSpec(memory_space=pltpu.SMEM)] * 2),
        out_specs=pl.BlockSpec(memory_space=pltpu.VMEM),
        scratch_shapes=[
            pltpu.VMEM((N_DEV, TOK, D), jnp.bfloat16),
            pltpu.VMEM((N_DEV, TOK, D), jnp.float32),
            pltpu.VMEM((2, TOK, D), jnp.bfloat16),
            pltpu.VMEM((2, TOK, D), jnp.bfloat16),
            pltpu.SemaphoreType.DMA((2,)),
            pltpu.SemaphoreType.DMA((2,)),
            pltpu.SemaphoreType.DMA((2,)),
            pltpu.SemaphoreType.DMA((2,)),
            pltpu.SemaphoreType.DMA((2,)),
            pltpu.SemaphoreType.DMA((2,)),
            pltpu.SemaphoreType.DMA((2,)),
            pltpu.SemaphoreType.DMA((2,)),
        ],
        compiler_params=pltpu.CompilerParams(collective_id=0),
    )(xs, wq, wk, wv, wo,
      jnp.asarray(_COS), jnp.asarray(_SIN),
      jnp.asarray(_R256, dtype=jnp.bfloat16), jnp.asarray(_MASK),
      jnp.asarray(_PERM), jnp.asarray(_RANK))
    return out.reshape(B_LOC, SQ, D)


# device time: 153404 ns/iter; 3.2257x vs baseline; 1.0285x over previous
import numpy as np
import jax
import jax.numpy as jnp
from jax import lax
from jax.experimental import pallas as pl
from jax.experimental.pallas import tpu as pltpu

N_DEV = 32
B_LOC = 2
SQ = 128
D = 512
H_LOC = 4
DH = 64
TOK = B_LOC * SQ
PAIR = 2 * TOK

_inv = 1.0 / (10000.0 ** (np.arange(0, DH, 2) / DH))
_pos = np.arange(SQ)[:, None] * _inv[None, :]
_cos = np.repeat(np.cos(_pos), 2, axis=-1)
_sin = np.repeat(np.sin(_pos), 2, axis=-1)
_COS = np.tile(_cos, (2 * B_LOC, H_LOC)).astype(np.float32)
_SIN = np.tile(_sin, (2 * B_LOC, H_LOC)).astype(np.float32)
_R = np.zeros((DH, DH), dtype=np.float32)
for _k in range(DH // 2):
    _R[2 * _k + 1, 2 * _k] = -1.0
    _R[2 * _k, 2 * _k + 1] = 1.0
_R256 = np.kron(np.eye(H_LOC, dtype=np.float32), _R)
_MASK = np.full((PAIR, PAIR), -1e9, dtype=np.float32)
for _b in range(2 * B_LOC):
    _MASK[_b * SQ:(_b + 1) * SQ, _b * SQ:(_b + 1) * SQ] = 0.0

def _logical(x, y, z):
    return z * 8 + 2 * y + (x if y % 2 == 0 else 1 - x)

_w = []
for _y in range(4):
    for _z in (range(4) if _y % 2 == 0 else range(3, -1, -1)):
        _w.append((_y, _z))
_cycle = [(0, y, z) for (y, z) in _w] + [(1, y, z) for (y, z) in reversed(_w)]
_PERM = np.array([_logical(x, y, z) for (x, y, z) in _cycle], dtype=np.int32)
_RANK = np.zeros(N_DEV, dtype=np.int32)
for _r, _p in enumerate(_PERM):
    _RANK[_p] = _r


def kernel(x, Wq, Wk, Wv, Wo):
    xs = x.astype(jnp.bfloat16).reshape(TOK, D)
    wq = Wq.astype(jnp.bfloat16)
    wk = Wk.astype(jnp.bfloat16)
    wv = Wv.astype(jnp.bfloat16)
    wo = Wo.astype(jnp.bfloat16)

    def body(x_ref, wq_ref, wk_ref, wv_ref, wo_ref, cos_ref, sin_ref, r_ref,
             mask_ref, perm_ref, rank_ref, out_ref, xfull, part,
             comm_r, comm_l, macc, min_buf,
             dir_send, dir_recv, agr_s, agr_r, agl_s, agl_r,
             rsr_s, rsr_r, rsl_s, rsl_r):
        def mod(v):
            return lax.rem(v + 4 * N_DEV, N_DEV)

        def bstart(v):
            return lax.rem(mod(v), 8) * 4

        def slot(v):
            return lax.rem(mod(v), 8) * 4 + lax.div(mod(v), 8)

        me = rank_ref[lax.axis_index("i")]
        right = perm_ref[mod(me + 1)]
        left = perm_ref[mod(me - 1)]

        barrier_sem = pltpu.get_barrier_semaphore()
        for nbr in (left, right,
                    perm_ref[mod(me + 8)], perm_ref[mod(me - 8)],
                    perm_ref[mod(me + 16)]):
            pl.semaphore_signal(
                barrier_sem, inc=1,
                device_id=(nbr,), device_id_type=pl.DeviceIdType.MESH,
            )
        pl.semaphore_wait(barrier_sem, 5)

        slot_me = slot(me)
        xfull[slot_me] = x_ref[:, :]

        cos_c = cos_ref[:, :]
        sin_c = sin_ref[:, :]
        r256 = r_ref[:, :]
        mask_c = mask_ref[:, :]

        def attention(xa):
            q = jnp.dot(xa, wq_ref[:, :], preferred_element_type=jnp.float32)
            k = jnp.dot(xa, wk_ref[:, :], preferred_element_type=jnp.float32)
            v = jnp.dot(xa, wv_ref[:, :],
                        preferred_element_type=jnp.float32).astype(jnp.bfloat16)
            qr = jnp.dot(q.astype(jnp.bfloat16), r256,
                         preferred_element_type=jnp.float32)
            kr = jnp.dot(k.astype(jnp.bfloat16), r256,
                         preferred_element_type=jnp.float32)
            qrot = (q * cos_c + qr * sin_c).astype(jnp.bfloat16)
            krot = (k * cos_c + kr * sin_c).astype(jnp.bfloat16)
            ctxs = []
            for hh in range(H_LOC):
                c0 = hh * DH
                qh = qrot[:, c0:c0 + DH]
                kh = krot[:, c0:c0 + DH]
                vh = v[:, c0:c0 + DH]
                s = lax.dot_general(
                    qh, kh, (((1,), (1,)), ((), ())),
                    preferred_element_type=jnp.float32) * 0.125 + mask_c
                e = jnp.exp(s)
                w = e / jnp.sum(e, axis=-1, keepdims=True)
                ctxs.append(jnp.dot(w.astype(jnp.bfloat16), vh,
                                    preferred_element_type=jnp.float32))
            ctx = jnp.concatenate(ctxs, axis=1).astype(jnp.bfloat16)
            return jnp.dot(ctx, wo_ref[:, :], preferred_element_type=jnp.float32)

        def compute_pair(s1, s2):
            xa = jnp.concatenate([xfull[s1], xfull[s2]], axis=0)
            res = attention(xa).astype(jnp.bfloat16)
            part[s1] = res[:TOK]
            part[s2] = res[TOK:]

        def compute_block(cid):
            bs = bstart(cid)

            def one(i, car):
                compute_pair(bs + 2 * i, bs + 2 * i + 1)
                return car

            lax.fori_loop(0, 2, one, 0)

        for k in (1, 2, 3):
            sd = pltpu.make_async_remote_copy(
                src_ref=xfull.at[slot_me], dst_ref=xfull.at[slot_me],
                send_sem=dir_send.at[k - 1], recv_sem=dir_recv.at[k - 1],
                device_id=(perm_ref[mod(me + 8 * k)],),
                device_id_type=pl.DeviceIdType.MESH,
            )
            sd.start()
        seed_ins = []
        for k in (1, 2, 3):
            rc = pltpu.make_async_remote_copy(
                src_ref=xfull.at[slot_me], dst_ref=xfull.at[slot(me - 8 * k)],
                send_sem=dir_send.at[k - 1], recv_sem=dir_recv.at[k - 1],
                device_id=(me,), device_id_type=pl.DeviceIdType.MESH,
            )
            rc.wait_recv()
            seed_ins.append(rc)
        for k in (1, 2, 3):
            sd = pltpu.make_async_remote_copy(
                src_ref=xfull.at[slot_me], dst_ref=xfull.at[slot_me],
                send_sem=dir_send.at[k - 1], recv_sem=dir_recv.at[k - 1],
                device_id=(me,), device_id_type=pl.DeviceIdType.MESH,
            )
            sd.wait_send()

        for h in range(4):
            br = bstart(me - h)
            rr = pltpu.make_async_remote_copy(
                src_ref=xfull.at[pl.ds(br, 4)], dst_ref=xfull.at[pl.ds(br, 4)],
                send_sem=agr_s.at[h % 2], recv_sem=agr_r.at[h % 2],
                device_id=(right,), device_id_type=pl.DeviceIdType.MESH,
            )
            rr.start()
            ll = None
            if h < 3:
                bl = bstart(me + h)
                ll = pltpu.make_async_remote_copy(
                    src_ref=xfull.at[pl.ds(bl, 4)],
                    dst_ref=xfull.at[pl.ds(bl, 4)],
                    send_sem=agl_s.at[h % 2], recv_sem=agl_r.at[h % 2],
                    device_id=(left,), device_id_type=pl.DeviceIdType.MESH,
                )
                ll.start()
            if h == 0:
                compute_block(me)
            else:
                compute_block(me - h)
                compute_block(me + h)
            rr.wait()
            if ll is not None:
                ll.wait()
        compute_block(me - 4)

        bpr = bstart(me + 4)
        bpl = bstart(me - 3)
        for j in range(4):
            comm_r[0, j] = part[bpr + j]
            comm_l[0, j] = part[bpl + j]
        for s in range(4):
            snd = s % 2
            rcv = (s + 1) % 2
            rr = pltpu.make_async_remote_copy(
                src_ref=comm_r.at[snd], dst_ref=comm_r.at[rcv],
                send_sem=rsr_s.at[snd], recv_sem=rsr_r.at[rcv],
                device_id=(right,), device_id_type=pl.DeviceIdType.MESH,
            )
            rr.start()
            if s < 3:
                ll = pltpu.make_async_remote_copy(
                    src_ref=comm_l.at[snd], dst_ref=comm_l.at[rcv],
                    send_sem=rsl_s.at[snd], recv_sem=rsl_r.at[rcv],
                    device_id=(left,), device_id_type=pl.DeviceIdType.MESH,
                )
                ll.start()
                ll.wait()
                if s < 2:
                    bl = bstart(me - 2 + s)
                    for j in range(4):
                        comm_l[rcv, j] = (
                            comm_l[rcv, j].astype(jnp.float32)
                            + part[bl + j].astype(jnp.float32)
                        ).astype(jnp.bfloat16)
            rr.wait()
            if s < 3:
                br = bstart(me + 3 - s)
                for j in range(4):
                    comm_r[rcv, j] = (
                        comm_r[rcv, j].astype(jnp.float32)
                        + part[br + j].astype(jnp.float32)
                    ).astype(jnp.bfloat16)

        bme = bstart(me)
        for j in range(4):
            macc[j] = (comm_r[0, j].astype(jnp.float32)
                       + comm_l[1, j].astype(jnp.float32)
                       + part[bme + j].astype(jnp.float32)).astype(jnp.bfloat16)
        for k in (1, 2, 3):
            jk = lax.rem(lax.div(mod(me), 8) - k + 8, 4)
            lr = pltpu.make_async_remote_copy(
                src_ref=macc.at[jk], dst_ref=min_buf.at[k - 1],
                send_sem=dir_send.at[k - 1], recv_sem=dir_recv.at[k - 1],
                device_id=(perm_ref[mod(me - 8 * k)],),
                device_id_type=pl.DeviceIdType.MESH,
            )
            lr.start()
        for k in (1, 2, 3):
            rc = pltpu.make_async_remote_copy(
                src_ref=macc.at[0], dst_ref=min_buf.at[k - 1],
                send_sem=dir_send.at[k - 1], recv_sem=dir_recv.at[k - 1],
                device_id=(me,), device_id_type=pl.DeviceIdType.MESH,
            )
            rc.wait_recv()
        for k in (1, 2, 3):
            sd = pltpu.make_async_remote_copy(
                src_ref=macc.at[0], dst_ref=min_buf.at[k - 1],
                send_sem=dir_send.at[k - 1], recv_sem=dir_recv.at[k - 1],
                device_id=(me,), device_id_type=pl.DeviceIdType.MESH,
            )
            sd.wait_send()

        j0 = lax.div(mod(me), 8)
        out_ref[:, :] = (macc[j0].astype(jnp.float32)
                         + min_buf[0].astype(jnp.float32)
                         + min_buf[1].astype(jnp.float32)
                         + min_buf[2].astype(jnp.float32))

    out = pl.pallas_call(
        body,
        out_shape=jax.ShapeDtypeStruct((TOK, D), jnp.float32),
        in_specs=([pl.BlockSpec(memory_space=pltpu.VMEM)] * 9
                  + [pl.BlockSpec(memory_space=pltpu.SMEM)] * 2),
        out_specs=pl.BlockSpec(memory_space=pltpu.VMEM),
        scratch_shapes=[
            pltpu.VMEM((N_DEV, TOK, D), jnp.bfloat16),
            pltpu.VMEM((N_DEV, TOK, D), jnp.bfloat16),
            pltpu.VMEM((2, 4, TOK, D), jnp.bfloat16),
            pltpu.VMEM((2, 4, TOK, D), jnp.bfloat16),
            pltpu.VMEM((4, TOK, D), jnp.bfloat16),
            pltpu.VMEM((3, TOK, D), jnp.bfloat16),
            pltpu.SemaphoreType.DMA((3,)),
            pltpu.SemaphoreType.DMA((3,)),
            pltpu.SemaphoreType.DMA((2,)),
            pltpu.SemaphoreType.DMA((2,)),
            pltpu.SemaphoreType.DMA((2,)),
            pltpu.SemaphoreType.DMA((2,)),
            pltpu.SemaphoreType.DMA((2,)),
            pltpu.SemaphoreType.DMA((2,)),
            pltpu.SemaphoreType.DMA((2,)),
            pltpu.SemaphoreType.DMA((2,)),
        ],
        compiler_params=pltpu.CompilerParams(
            collective_id=0, vmem_limit_bytes=64 * 1024 * 1024),
    )(xs, wq, wk, wv, wo,
      jnp.asarray(_COS), jnp.asarray(_SIN),
      jnp.asarray(_R256, dtype=jnp.bfloat16), jnp.asarray(_MASK),
      jnp.asarray(_PERM), jnp.asarray(_RANK))
    return out.reshape(B_LOC, SQ, D)


# device time: 152983 ns/iter; 3.2345x vs baseline; 1.0028x over previous
import numpy as np
import jax
import jax.numpy as jnp
from jax import lax
from jax.experimental import pallas as pl
from jax.experimental.pallas import tpu as pltpu

N_DEV = 32
B_LOC = 2
SQ = 128
D = 512
H_LOC = 4
DH = 64
TOK = B_LOC * SQ
PAIR = 2 * TOK

_inv = 1.0 / (10000.0 ** (np.arange(0, DH, 2) / DH))
_pos = np.arange(SQ)[:, None] * _inv[None, :]
_cos = np.repeat(np.cos(_pos), 2, axis=-1)
_sin = np.repeat(np.sin(_pos), 2, axis=-1)
_COS = np.tile(_cos, (2 * B_LOC, H_LOC)).astype(np.float32)
_SIN = np.tile(_sin, (2 * B_LOC, H_LOC)).astype(np.float32)
_R = np.zeros((DH, DH), dtype=np.float32)
for _k in range(DH // 2):
    _R[2 * _k + 1, 2 * _k] = -1.0
    _R[2 * _k, 2 * _k + 1] = 1.0
_R256 = np.kron(np.eye(H_LOC, dtype=np.float32), _R)
_MASK = np.full((PAIR, PAIR), -1e9, dtype=np.float32)
for _b in range(2 * B_LOC):
    _MASK[_b * SQ:(_b + 1) * SQ, _b * SQ:(_b + 1) * SQ] = 0.0

def _logical(x, y, z):
    return z * 8 + 2 * y + (x if y % 2 == 0 else 1 - x)

_w = []
for _y in range(4):
    for _z in (range(4) if _y % 2 == 0 else range(3, -1, -1)):
        _w.append((_y, _z))
_cycle = [(0, y, z) for (y, z) in _w] + [(1, y, z) for (y, z) in reversed(_w)]
_PERM = np.array([_logical(x, y, z) for (x, y, z) in _cycle], dtype=np.int32)
_RANK = np.zeros(N_DEV, dtype=np.int32)
for _r, _p in enumerate(_PERM):
    _RANK[_p] = _r


def kernel(x, Wq, Wk, Wv, Wo):
    xs = x.astype(jnp.bfloat16).reshape(TOK, D)
    wq = Wq.astype(jnp.bfloat16)
    wk = Wk.astype(jnp.bfloat16)
    wv = Wv.astype(jnp.bfloat16)
    wo = Wo.astype(jnp.bfloat16)

    def body(x_ref, wq_ref, wk_ref, wv_ref, wo_ref, cos_ref, sin_ref, r_ref,
             mask_ref, perm_ref, rank_ref, out_ref, xfull, part,
             comm_r, comm_l, macc, min_buf,
             dir_send, dir_recv, agr_s, agr_r, agl_s, agl_r,
             rsr_s, rsr_r, rsl_s, rsl_r):
        def mod(v):
            return lax.rem(v + 4 * N_DEV, N_DEV)

        def bstart(v):
            return lax.rem(mod(v), 8) * 4

        def slot(v):
            return lax.rem(mod(v), 8) * 4 + lax.div(mod(v), 8)

        me = rank_ref[lax.axis_index("i")]
        right = perm_ref[mod(me + 1)]
        left = perm_ref[mod(me - 1)]

        barrier_sem = pltpu.get_barrier_semaphore()
        for nbr in (left, right,
                    perm_ref[mod(me + 8)], perm_ref[mod(me - 8)],
                    perm_ref[mod(me + 16)]):
            pl.semaphore_signal(
                barrier_sem, inc=1,
                device_id=(nbr,), device_id_type=pl.DeviceIdType.MESH,
            )
        pl.semaphore_wait(barrier_sem, 5)

        slot_me = slot(me)
        xfull[slot_me] = x_ref[:, :]

        cos_c = cos_ref[:, :]
        sin_c = sin_ref[:, :]
        r256 = r_ref[:, :]
        mask_c = mask_ref[:, :]

        def attention(xa):
            q = jnp.dot(xa, wq_ref[:, :], preferred_element_type=jnp.float32)
            k = jnp.dot(xa, wk_ref[:, :], preferred_element_type=jnp.float32)
            v = jnp.dot(xa, wv_ref[:, :],
                        preferred_element_type=jnp.float32).astype(jnp.bfloat16)
            qr = jnp.dot(q.astype(jnp.bfloat16), r256,
                         preferred_element_type=jnp.float32)
            kr = jnp.dot(k.astype(jnp.bfloat16), r256,
                         preferred_element_type=jnp.float32)
            qrot = (q * cos_c + qr * sin_c).astype(jnp.bfloat16)
            krot = (k * cos_c + kr * sin_c).astype(jnp.bfloat16)
            ctxs = []
            for hh in range(H_LOC):
                c0 = hh * DH
                qh = qrot[:, c0:c0 + DH]
                kh = krot[:, c0:c0 + DH]
                vh = v[:, c0:c0 + DH]
                s = lax.dot_general(
                    qh, kh, (((1,), (1,)), ((), ())),
                    preferred_element_type=jnp.float32) * 0.125 + mask_c
                e = jnp.exp(s)
                w = e / jnp.sum(e, axis=-1, keepdims=True)
                ctxs.append(jnp.dot(w.astype(jnp.bfloat16), vh,
                                    preferred_element_type=jnp.float32))
            ctx = jnp.concatenate(ctxs, axis=1).astype(jnp.bfloat16)
            return jnp.dot(ctx, wo_ref[:, :], preferred_element_type=jnp.float32)

        def compute_pair(s1, s2):
            xa = jnp.concatenate([xfull[s1], xfull[s2]], axis=0)
            res = attention(xa).astype(jnp.bfloat16)
            part[s1] = res[:TOK]
            part[s2] = res[TOK:]

        def compute_block(cid):
            bs = bstart(cid)

            def one(i, car):
                compute_pair(bs + 2 * i, bs + 2 * i + 1)
                return car

            lax.fori_loop(0, 2, one, 0)

        for k in (1, 2, 3):
            sd = pltpu.make_async_remote_copy(
                src_ref=xfull.at[slot_me], dst_ref=xfull.at[slot_me],
                send_sem=dir_send.at[k - 1], recv_sem=dir_recv.at[k - 1],
                device_id=(perm_ref[mod(me + 8 * k)],),
                device_id_type=pl.DeviceIdType.MESH,
            )
            sd.start()
        seed_ins = []
        for k in (1, 2, 3):
            rc = pltpu.make_async_remote_copy(
                src_ref=xfull.at[slot_me], dst_ref=xfull.at[slot(me - 8 * k)],
                send_sem=dir_send.at[k - 1], recv_sem=dir_recv.at[k - 1],
                device_id=(me,), device_id_type=pl.DeviceIdType.MESH,
            )
            rc.wait_recv()
            seed_ins.append(rc)
        for k in (1, 2, 3):
            sd = pltpu.make_async_remote_copy(
                src_ref=xfull.at[slot_me], dst_ref=xfull.at[slot_me],
                send_sem=dir_send.at[k - 1], recv_sem=dir_recv.at[k - 1],
                device_id=(me,), device_id_type=pl.DeviceIdType.MESH,
            )
            sd.wait_send()

        for h in range(4):
            br = bstart(me - h)
            rr = pltpu.make_async_remote_copy(
                src_ref=xfull.at[pl.ds(br, 4)], dst_ref=xfull.at[pl.ds(br, 4)],
                send_sem=agr_s.at[h % 2], recv_sem=agr_r.at[h % 2],
                device_id=(right,), device_id_type=pl.DeviceIdType.MESH,
            )
            rr.start()
            ll = None
            if h < 3:
                bl = bstart(me + h)
                ll = pltpu.make_async_remote_copy(
                    src_ref=xfull.at[pl.ds(bl, 4)],
                    dst_ref=xfull.at[pl.ds(bl, 4)],
                    send_sem=agl_s.at[h % 2], recv_sem=agl_r.at[h % 2],
                    device_id=(left,), device_id_type=pl.DeviceIdType.MESH,
                )
                ll.start()
            if h == 0:
                compute_block(me)
            else:
                compute_block(me - h)
                compute_block(me + h)
            rr.wait()
            if ll is not None:
                ll.wait()

        def pblock(b):
            return part[pl.ds(b, 4)].reshape(4 * TOK, D)

        def rs_rdma(comm, snd, rcv, ssem, rsem, dev):
            return pltpu.make_async_remote_copy(
                src_ref=comm.at[snd], dst_ref=comm.at[rcv],
                send_sem=ssem.at[snd], recv_sem=rsem.at[rcv],
                device_id=(dev,), device_id_type=pl.DeviceIdType.MESH,
            )

        def acc_into(comm, slotidx, b):
            comm[slotidx] = (comm[slotidx].astype(jnp.float32)
                             + pblock(b).astype(jnp.float32)
                             ).astype(jnp.bfloat16)

        comm_l[0] = pblock(bstart(me - 3))
        ll0 = rs_rdma(comm_l, 0, 1, rsl_s, rsl_r, left)
        ll0.start()
        compute_block(me - 4)
        comm_r[0] = pblock(bstart(me + 4))
        rr0 = rs_rdma(comm_r, 0, 1, rsr_s, rsr_r, right)
        rr0.start()
        ll0.wait()
        acc_into(comm_l, 1, bstart(me - 2))
        ll1 = rs_rdma(comm_l, 1, 0, rsl_s, rsl_r, left)
        ll1.start()
        rr0.wait()
        acc_into(comm_r, 1, bstart(me + 3))
        rr1 = rs_rdma(comm_r, 1, 0, rsr_s, rsr_r, right)
        rr1.start()
        ll1.wait()
        acc_into(comm_l, 0, bstart(me - 1))
        ll2 = rs_rdma(comm_l, 0, 1, rsl_s, rsl_r, left)
        ll2.start()
        rr1.wait()
        acc_into(comm_r, 0, bstart(me + 2))
        rr2 = rs_rdma(comm_r, 0, 1, rsr_s, rsr_r, right)
        rr2.start()
        ll2.wait()
        rr2.wait()
        acc_into(comm_r, 1, bstart(me + 1))
        rr3 = rs_rdma(comm_r, 1, 0, rsr_s, rsr_r, right)
        rr3.start()
        rr3.wait()

        macc[:, :] = (comm_r[0].astype(jnp.float32)
                      + comm_l[1].astype(jnp.float32)
                      + pblock(bstart(me)).astype(jnp.float32)
                      ).astype(jnp.bfloat16)
        for k in (1, 2, 3):
            jk = lax.rem(lax.div(mod(me), 8) - k + 8, 4)
            lr = pltpu.make_async_remote_copy(
                src_ref=macc.at[pl.ds(jk * TOK, TOK)],
                dst_ref=min_buf.at[k - 1],
                send_sem=dir_send.at[k - 1], recv_sem=dir_recv.at[k - 1],
                device_id=(perm_ref[mod(me - 8 * k)],),
                device_id_type=pl.DeviceIdType.MESH,
            )
            lr.start()
        for k in (1, 2, 3):
            rc = pltpu.make_async_remote_copy(
                src_ref=macc.at[pl.ds(0, TOK)], dst_ref=min_buf.at[k - 1],
                send_sem=dir_send.at[k - 1], recv_sem=dir_recv.at[k - 1],
                device_id=(me,), device_id_type=pl.DeviceIdType.MESH,
            )
            rc.wait_recv()
        for k in (1, 2, 3):
            sd = pltpu.make_async_remote_copy(
                src_ref=macc.at[pl.ds(0, TOK)], dst_ref=min_buf.at[k - 1],
                send_sem=dir_send.at[k - 1], recv_sem=dir_recv.at[k - 1],
                device_id=(me,), device_id_type=pl.DeviceIdType.MESH,
            )
            sd.wait_send()

        j0 = lax.div(mod(me), 8)
        out_ref[:, :] = (macc[pl.ds(j0 * TOK, TOK)].astype(jnp.float32)
                         + min_buf[0].astype(jnp.float32)
                         + min_buf[1].astype(jnp.float32)
                         + min_buf[2].astype(jnp.float32))

    out = pl.pallas_call(
        body,
        out_shape=jax.ShapeDtypeStruct((TOK, D), jnp.float32),
        in_specs=([pl.BlockSpec(memory_space=pltpu.VMEM)] * 9
                  + [pl.BlockSpec(memory_space=pltpu.SMEM)] * 2),
        out_specs=pl.BlockSpec(memory_space=pltpu.VMEM),
        scratch_shapes=[
            pltpu.VMEM((N_DEV, TOK, D), jnp.bfloat16),
            pltpu.VMEM((N_DEV, TOK, D), jnp.bfloat16),
            pltpu.VMEM((2, 4 * TOK, D), jnp.bfloat16),
            pltpu.VMEM((2, 4 * TOK, D), jnp.bfloat16),
            pltpu.VMEM((4 * TOK, D), jnp.bfloat16),
            pltpu.VMEM((3, TOK, D), jnp.bfloat16),
            pltpu.SemaphoreType.DMA((3,)),
            pltpu.SemaphoreType.DMA((3,)),
            pltpu.SemaphoreType.DMA((2,)),
            pltpu.SemaphoreType.DMA((2,)),
            pltpu.SemaphoreType.DMA((2,)),
            pltpu.SemaphoreType.DMA((2,)),
            pltpu.SemaphoreType.DMA((2,)),
            pltpu.SemaphoreType.DMA((2,)),
            pltpu.SemaphoreType.DMA((2,)),
            pltpu.SemaphoreType.DMA((2,)),
        ],
        compiler_params=pltpu.CompilerParams(
            collective_id=0, vmem_limit_bytes=64 * 1024 * 1024),
    )(xs, wq, wk, wv, wo,
      jnp.asarray(_COS), jnp.asarray(_SIN),
      jnp.asarray(_R256, dtype=jnp.bfloat16), jnp.asarray(_MASK),
      jnp.asarray(_PERM), jnp.asarray(_RANK))
    return out.reshape(B_LOC, SQ, D)
